# baseline (speedup 1.0000x reference)
import functools

import jax
import jax.numpy as jnp
from jax import lax
from jax.experimental import pallas as pl
from jax.experimental.pallas import tpu as pltpu
from jax.experimental.pallas import tpu_sc as plsc

_PATCH = 8
_IDX_CHUNK = 128


@functools.lru_cache(maxsize=None)
def _make_sc_gather(num_idx: int, dim: int):
    info = plsc.get_sparse_core_info()
    nc, ns = info.num_cores, info.num_subcores
    nw = nc * ns
    rows_per_w = num_idx // nw
    chunks = rows_per_w // _IDX_CHUNK
    mesh = plsc.VectorSubcoreMesh(core_axis_name="c", subcore_axis_name="s")

    @functools.partial(
        pl.kernel,
        mesh=mesh,
        out_type=jax.ShapeDtypeStruct((num_idx, dim), jnp.float32),
        scratch_types=[
            pltpu.VMEM((chunks, _IDX_CHUNK), jnp.int32),
            pltpu.VMEM((rows_per_w, dim), jnp.float32),
            pltpu.SemaphoreType.DMA,
        ],
        compiler_params=pltpu.CompilerParams(use_tc_tiling_on_sc=False),
    )
    def gather(idx_hbm, table_hbm, out_hbm, idx_v, rows_v, sem):
        wid = lax.axis_index("s") * nc + lax.axis_index("c")
        pltpu.sync_copy(idx_hbm.at[pl.ds(wid * chunks, chunks)], idx_v)
        copies = []
        for ci in range(chunks):
            copies.append(
                pltpu.async_copy(
                    table_hbm.at[idx_v.at[ci]],
                    rows_v.at[pl.ds(ci * _IDX_CHUNK, _IDX_CHUNK)],
                    sem,
                )
            )
        for cp in copies:
            cp.wait()
        pltpu.sync_copy(rows_v, out_hbm.at[pl.ds(wid * rows_per_w, rows_per_w)])

    return gather


def _mm_body(a0_ref, a1_ref, w0_ref, w1_ref, b_ref, o_ref):
    o_ref[...] = (
        jnp.dot(a0_ref[0], w0_ref[0], preferred_element_type=jnp.float32)
        + jnp.dot(a1_ref[0], w1_ref[0], preferred_element_type=jnp.float32)
        + b_ref[...]
    )


def _tc_matmul(a, w2, b2d, bm):
    _, m, k = a.shape
    n = w2.shape[2]
    return pl.pallas_call(
        _mm_body,
        grid=(m // bm,),
        in_specs=[
            pl.BlockSpec((1, bm, k), lambda i: (0, i, 0)),
            pl.BlockSpec((1, bm, k), lambda i: (1, i, 0)),
            pl.BlockSpec((1, k, n), lambda i: (0, 0, 0)),
            pl.BlockSpec((1, k, n), lambda i: (1, 0, 0)),
            pl.BlockSpec((1, n), lambda i: (0, 0)),
        ],
        out_specs=pl.BlockSpec((bm, n), lambda i: (i, 0)),
        out_shape=jax.ShapeDtypeStruct((m, n), jnp.float32),
        compiler_params=pltpu.CompilerParams(
            dimension_semantics=("arbitrary",),
        ),
    )(a, a, w2, w2, b2d)


def kernel(bytes_flat, table, W, b):
    B, L = bytes_flat.shape
    P = _PATCH
    T = L // P
    byte_dim = table.shape[1]
    n_idx = B * T * P
    half = P * byte_dim // 2

    patches = bytes_flat[:, : T * P].reshape(B * T, 2, P // 2)
    idx_perm = patches.transpose(1, 0, 2).reshape(
        n_idx // _IDX_CHUNK, _IDX_CHUNK
    )

    gather = _make_sc_gather(n_idx, byte_dim)
    embs = gather(idx_perm, table)

    a = embs.reshape(2, B * T, half)
    w2 = W.reshape(2, half, -1)
    out = _tc_matmul(a, w2, b.reshape(1, -1), 512)
    return out.reshape(B, T, -1), T


# speedup vs baseline: 1.6836x; 1.6836x over previous
import functools

import jax
import jax.numpy as jnp
from jax import lax
from jax.experimental import pallas as pl
from jax.experimental.pallas import tpu as pltpu
from jax.experimental.pallas import tpu_sc as plsc

_PATCH = 8
_IDX_CHUNK = 128


@functools.lru_cache(maxsize=None)
def _make_sc_gather(num_idx: int, dim: int):
    info = plsc.get_sparse_core_info()
    nc, ns = info.num_cores, info.num_subcores
    nw = nc * ns
    rows_per_w = num_idx // nw
    chunks = rows_per_w // _IDX_CHUNK
    mesh = plsc.VectorSubcoreMesh(core_axis_name="c", subcore_axis_name="s")

    @functools.partial(
        pl.kernel,
        mesh=mesh,
        out_type=jax.ShapeDtypeStruct((num_idx, dim), jnp.float32),
        scratch_types=[
            pltpu.VMEM((chunks, _IDX_CHUNK), jnp.int32),
            pltpu.VMEM((rows_per_w, dim), jnp.float32),
            pltpu.SemaphoreType.DMA,
        ],
        compiler_params=pltpu.CompilerParams(use_tc_tiling_on_sc=False),
    )
    def gather(idx_hbm, table_hbm, out_hbm, idx_v, rows_v, sem):
        wid = lax.axis_index("s") * nc + lax.axis_index("c")
        pltpu.sync_copy(idx_hbm.at[pl.ds(wid * chunks, chunks)], idx_v)
        copies = []
        for ci in range(chunks):
            copies.append(
                pltpu.async_copy(
                    table_hbm.at[idx_v.at[ci]],
                    rows_v.at[pl.ds(ci * _IDX_CHUNK, _IDX_CHUNK)],
                    sem,
                )
            )
        for cp in copies:
            cp.wait()
        pltpu.sync_copy(rows_v, out_hbm.at[pl.ds(wid * rows_per_w, rows_per_w)])

    return gather


def _mm_body(m_ref, w0_ref, w1_ref, b_ref, o_ref):
    bm = o_ref.shape[0]
    m3 = m_ref[...].reshape(bm, 2, 128)
    a0 = m3[:, 0, :]
    a1 = m3[:, 1, :]
    o_ref[...] = (
        jnp.dot(a0, w0_ref[0], preferred_element_type=jnp.float32)
        + jnp.dot(a1, w1_ref[0], preferred_element_type=jnp.float32)
        + b_ref[...][None, :]
    )


def _tc_matmul(m2d, w2, b, bm):
    m = m2d.shape[0] // 2
    n = w2.shape[2]
    return pl.pallas_call(
        _mm_body,
        grid=(m // bm,),
        in_specs=[
            pl.BlockSpec((2 * bm, 128), lambda i: (i, 0)),
            pl.BlockSpec((1, 128, n), lambda i: (0, 0, 0)),
            pl.BlockSpec((1, 128, n), lambda i: (1, 0, 0)),
            pl.BlockSpec((n,), lambda i: (0,)),
        ],
        out_specs=pl.BlockSpec((bm, n), lambda i: (i, 0)),
        out_shape=jax.ShapeDtypeStruct((m, n), jnp.float32),
        compiler_params=pltpu.CompilerParams(
            dimension_semantics=("arbitrary",),
        ),
    )(m2d, w2, w2, b)


def kernel(bytes_flat, table, W, b):
    B, L = bytes_flat.shape
    P = _PATCH
    T = L // P
    byte_dim = table.shape[1]
    n_idx = B * T * P
    half = P * byte_dim // 2

    idx2d = bytes_flat[:, : T * P].reshape(n_idx // _IDX_CHUNK, _IDX_CHUNK)
    gather = _make_sc_gather(n_idx, byte_dim)
    embs = gather(idx2d, table)

    m2d = embs.reshape(n_idx * byte_dim // half, half)
    w2 = W.reshape(2, half, -1)
    out = _tc_matmul(m2d, w2, b, 512)
    return out.reshape(B, T, -1), T
